# initial kernel scaffold (unmeasured)
import jax
import jax.numpy as jnp
from jax import lax
from jax.experimental import pallas as pl
from jax.experimental.pallas import tpu as pltpu

N_DEV = 4
E_LOCAL = 4
N_EXP = 16
M = 512
D = 256
H = 512
M_BLK = M // N_DEV


def kernel(x, router_W, route_idx, expert_W, shared_W):
    def body(x_ref, rw_ref, idx_ref, ew_ref, sw_ref, out_ref,
             p_ref, send_buf, recv_buf, send_sems, recv_sems):
        my = lax.axis_index("i")
        left = lax.rem(my - 1 + N_DEV, N_DEV)
        right = lax.rem(my + 1, N_DEV)

        barrier_sem = pltpu.get_barrier_semaphore()
        for nbr in (left, right):
            pl.semaphore_signal(
                barrier_sem, inc=1,
                device_id=(nbr,), device_id_type=pl.DeviceIdType.MESH,
            )
        pl.semaphore_wait(barrier_sem, 2)

        x_f32 = x_ref[:, :]
        scores = jnp.dot(x_f32, rw_ref[:, :], preferred_element_type=jnp.float32)
        s_max = jnp.max(scores, axis=-1, keepdims=True)
        e = jnp.exp(scores - s_max)
        probs = e / jnp.sum(e, axis=-1, keepdims=True)

        eids = lax.broadcasted_iota(jnp.int32, (M, N_EXP), 1)
        w_all = jnp.where(eids == idx_ref[:, :], probs, 0.0)

        base = my * E_LOCAL
        acc = jnp.zeros((M, H), jnp.float32)
        for j in range(E_LOCAL):
            wj = lax.dynamic_slice(w_all, (0, base + j), (M, 1))
            xj = (wj * x_f32).astype(jnp.bfloat16)
            acc = acc + jnp.dot(
                xj, ew_ref[j, :, :].astype(jnp.bfloat16),
                preferred_element_type=jnp.float32,
            )
        p_ref[:, :] = acc

        for h in range(N_DEV - 1):
            c = lax.rem(my - 1 - h + 2 * N_DEV, N_DEV)
            chunk = p_ref[pl.ds(c * M_BLK, M_BLK), :]
            if h == 0:
                send_buf[h, :, :] = chunk
            else:
                send_buf[h, :, :] = chunk + recv_buf[h - 1, :, :]
            rdma = pltpu.make_async_remote_copy(
                src_ref=send_buf.at[h],
                dst_ref=recv_buf.at[h],
                send_sem=send_sems.at[h],
                recv_sem=recv_sems.at[h],
                device_id=(right,),
                device_id_type=pl.DeviceIdType.MESH,
            )
            rdma.start()
            rdma.wait()

        x_blk = lax.dynamic_slice(x_f32, (my * M_BLK, 0), (M_BLK, D))
        shared = jnp.dot(
            x_blk.astype(jnp.bfloat16), sw_ref[:, :].astype(jnp.bfloat16),
            preferred_element_type=jnp.float32,
        )
        out_ref[:, :] = (
            recv_buf[N_DEV - 2, :, :]
            + p_ref[pl.ds(my * M_BLK, M_BLK), :]
            + shared
        )

    return pl.pallas_call(
        body,
        out_shape=jax.ShapeDtypeStruct((M_BLK, H), jnp.float32),
        in_specs=[pl.BlockSpec(memory_space=pltpu.VMEM)] * 5,
        out_specs=pl.BlockSpec(memory_space=pltpu.VMEM),
        scratch_shapes=[
            pltpu.VMEM((M, H), jnp.float32),
            pltpu.VMEM((N_DEV - 1, M_BLK, H), jnp.float32),
            pltpu.VMEM((N_DEV - 1, M_BLK, H), jnp.float32),
            pltpu.SemaphoreType.DMA((N_DEV - 1,)),
            pltpu.SemaphoreType.DMA((N_DEV - 1,)),
        ],
        compiler_params=pltpu.CompilerParams(collective_id=0),
    )(x, router_W, route_idx, expert_W, shared_W)


# baseline (device time: 24528 ns/iter reference)
import jax
import jax.numpy as jnp
from jax import lax
from jax.experimental import pallas as pl
from jax.experimental.pallas import tpu as pltpu

N_DEV = 4
E_LOCAL = 4
N_EXP = 16
M = 512
D = 256
H = 512
M_BLK = M // N_DEV


def kernel(x, router_W, route_idx, expert_W, shared_W):
    def body(x_ref, rw_ref, idx_ref, ew_ref, sw_ref, out_ref,
             p_ref, send_buf, recv_buf, send_sems, recv_sems):
        my = lax.axis_index("i")
        left = lax.rem(my - 1 + N_DEV, N_DEV)
        right = lax.rem(my + 1, N_DEV)

        barrier_sem = pltpu.get_barrier_semaphore()
        for nbr in (left, right):
            pl.semaphore_signal(
                barrier_sem, inc=1,
                device_id=(nbr,), device_id_type=pl.DeviceIdType.MESH,
            )
        pl.semaphore_wait(barrier_sem, 2)

        x_f32 = x_ref[:, :]
        scores = jnp.dot(x_f32, rw_ref[:, :], preferred_element_type=jnp.float32)
        s_max = jnp.max(scores, axis=-1, keepdims=True)
        e = jnp.exp(scores - s_max)
        probs = e / jnp.sum(e, axis=-1, keepdims=True)

        route = idx_ref[:, :]
        eids = lax.broadcasted_iota(jnp.int32, (M, N_EXP), 1)
        w_sel = jnp.sum(
            jnp.where(eids == route, probs, 0.0), axis=1, keepdims=True
        )

        base = my * E_LOCAL
        acc = jnp.zeros((M, H), jnp.float32)
        for j in range(E_LOCAL):
            wj = jnp.where(route == base + j, w_sel, 0.0)
            xj = (wj * x_f32).astype(jnp.bfloat16)
            acc = acc + jnp.dot(
                xj, ew_ref[j, :, :].astype(jnp.bfloat16),
                preferred_element_type=jnp.float32,
            )
        p_ref[:, :] = acc

        for h in range(N_DEV - 1):
            c = lax.rem(my - 1 - h + 2 * N_DEV, N_DEV)
            chunk = p_ref[pl.ds(c * M_BLK, M_BLK), :]
            if h == 0:
                send_buf[h, :, :] = chunk
            else:
                send_buf[h, :, :] = chunk + recv_buf[h - 1, :, :]
            rdma = pltpu.make_async_remote_copy(
                src_ref=send_buf.at[h],
                dst_ref=recv_buf.at[h],
                send_sem=send_sems.at[h],
                recv_sem=recv_sems.at[h],
                device_id=(right,),
                device_id_type=pl.DeviceIdType.MESH,
            )
            rdma.start()
            rdma.wait()

        x_blk = x_ref[pl.ds(my * M_BLK, M_BLK), :]
        shared = jnp.dot(
            x_blk.astype(jnp.bfloat16), sw_ref[:, :].astype(jnp.bfloat16),
            preferred_element_type=jnp.float32,
        )
        out_ref[:, :] = (
            recv_buf[N_DEV - 2, :, :]
            + p_ref[pl.ds(my * M_BLK, M_BLK), :]
            + shared
        )

    return pl.pallas_call(
        body,
        out_shape=jax.ShapeDtypeStruct((M_BLK, H), jnp.float32),
        in_specs=[pl.BlockSpec(memory_space=pltpu.VMEM)] * 5,
        out_specs=pl.BlockSpec(memory_space=pltpu.VMEM),
        scratch_shapes=[
            pltpu.VMEM((M, H), jnp.float32),
            pltpu.VMEM((N_DEV - 1, M_BLK, H), jnp.float32),
            pltpu.VMEM((N_DEV - 1, M_BLK, H), jnp.float32),
            pltpu.SemaphoreType.DMA((N_DEV - 1,)),
            pltpu.SemaphoreType.DMA((N_DEV - 1,)),
        ],
        compiler_params=pltpu.CompilerParams(collective_id=0),
    )(x, router_W, route_idx, expert_W, shared_W)


# device time: 15504 ns/iter; 1.5820x vs baseline; 1.5820x over previous
import jax
import jax.numpy as jnp
from jax import lax
from jax.experimental import pallas as pl
from jax.experimental.pallas import tpu as pltpu

N_DEV = 4
E_LOCAL = 4
N_EXP = 16
M = 512
D = 256
H = 512
M_BLK = M // N_DEV


def kernel(x, router_W, route_idx, expert_W, shared_W):
    def body(x_ref, rw_ref, idx_ref, ew_ref, sw_ref, out_ref,
             p_ref, pb_ref, recv_buf, send_sems, recv_sems):
        my = lax.axis_index("i")

        barrier_sem = pltpu.get_barrier_semaphore()
        for k in range(1, N_DEV):
            pl.semaphore_signal(
                barrier_sem, inc=1,
                device_id=(lax.rem(my + k, N_DEV),),
                device_id_type=pl.DeviceIdType.MESH,
            )
        pl.semaphore_wait(barrier_sem, N_DEV - 1)

        x_f32 = x_ref[:, :]
        scores = jnp.dot(x_f32, rw_ref[:, :], preferred_element_type=jnp.float32)
        s_max = jnp.max(scores, axis=-1, keepdims=True)
        e = jnp.exp(scores - s_max)
        probs = e / jnp.sum(e, axis=-1, keepdims=True)

        route = idx_ref[:, :]
        eids = lax.broadcasted_iota(jnp.int32, (M, N_EXP), 1)
        w_sel = jnp.sum(
            jnp.where(eids == route, probs, 0.0), axis=1, keepdims=True
        )

        base = my * E_LOCAL
        acc = jnp.zeros((M, H), jnp.float32)
        for j in range(E_LOCAL):
            wj = jnp.where(route == base + j, w_sel, 0.0)
            xj = (wj * x_f32).astype(jnp.bfloat16)
            acc = acc + jnp.dot(
                xj, ew_ref[j, :, :].astype(jnp.bfloat16),
                preferred_element_type=jnp.float32,
            )
        p_ref[:, :] = acc
        acc_bf = acc.astype(jnp.bfloat16)
        for p in range(N_DEV):
            pb_ref[p, :, :] = acc_bf[p * M_BLK:(p + 1) * M_BLK, :]

        sends = []
        for k in range(1, N_DEV):
            dest = lax.rem(my + k, N_DEV)
            rdma = pltpu.make_async_remote_copy(
                src_ref=pb_ref.at[dest],
                dst_ref=recv_buf.at[N_DEV - 1 - k],
                send_sem=send_sems.at[k - 1],
                recv_sem=recv_sems.at[N_DEV - 1 - k],
                device_id=(dest,),
                device_id_type=pl.DeviceIdType.MESH,
            )
            rdma.start()
            sends.append(rdma)

        x_blk = x_ref[pl.ds(my * M_BLK, M_BLK), :]
        shared = jnp.dot(
            x_blk.astype(jnp.bfloat16), sw_ref[:, :].astype(jnp.bfloat16),
            preferred_element_type=jnp.float32,
        )

        for r in range(N_DEV - 1):
            recv = pltpu.make_async_remote_copy(
                src_ref=pb_ref.at[0],
                dst_ref=recv_buf.at[r],
                send_sem=send_sems.at[0],
                recv_sem=recv_sems.at[r],
                device_id=(my,),
                device_id_type=pl.DeviceIdType.MESH,
            )
            recv.wait_recv()

        own = p_ref[pl.ds(my * M_BLK, M_BLK), :]
        total = own + shared
        for r in range(N_DEV - 1):
            total = total + recv_buf[r, :, :].astype(jnp.float32)
        out_ref[:, :] = total

        for rdma in sends:
            rdma.wait_send()

    return pl.pallas_call(
        body,
        out_shape=jax.ShapeDtypeStruct((M_BLK, H), jnp.float32),
        in_specs=[pl.BlockSpec(memory_space=pltpu.VMEM)] * 5,
        out_specs=pl.BlockSpec(memory_space=pltpu.VMEM),
        scratch_shapes=[
            pltpu.VMEM((M, H), jnp.float32),
            pltpu.VMEM((N_DEV, M_BLK, H), jnp.bfloat16),
            pltpu.VMEM((N_DEV - 1, M_BLK, H), jnp.bfloat16),
            pltpu.SemaphoreType.DMA((N_DEV - 1,)),
            pltpu.SemaphoreType.DMA((N_DEV - 1,)),
        ],
        compiler_params=pltpu.CompilerParams(collective_id=0),
    )(x, router_W, route_idx, expert_W, shared_W)
